# Initial kernel scaffold; baseline (speedup 1.0000x reference)
import jax
import jax.numpy as jnp
from jax.experimental import pallas as pl


def kernel(tet_v, surface_vid, surface_f):
    raise NotImplementedError("write your pallas kernel here")


# speedup vs baseline: 7.8133x; 7.8133x over previous
import jax
import jax.numpy as jnp
from jax import lax
from jax.experimental import pallas as pl
from jax.experimental.pallas import tpu as pltpu
from jax.experimental.pallas import tpu_sc as plsc

N_TET_V = 100000
NV = 50000
NF = 100000

NS = 16
L = 16

VC = 3200
NV_PAD = NS * VC
VB = VC // 128

FC = 6272
NF_PAD = NS * FC
FB = FC // 128

VC3 = NV_PAD // 32


def _body(tet_hbm, vid_hbm, f0_hbm, f1_hbm, f2_hbm, zer_hbm,
          px_o, py_o, pz_o, nx_o, ny_o, nz_o,
          vid_v, ix_v, iy_v, iz_v, px_v, py_v, pz_v,
          f0_v, f1_v, f2_v, b0_v, b1_v, b2_v,
          v0x, v0y, v0z, v1x, v1y, v1z, v2x, v2y, v2z,
          cx_v, cy_v, cz_v, ax_v, ay_v, az_v,
          spx, spy, spz, sax, say, saz, sem):
    c = lax.axis_index("c")
    s = lax.axis_index("s")

    for sh in (sax, say, saz):
        pltpu.sync_copy(zer_hbm, sh.at[pl.ds(s * VC, VC)])

    pltpu.sync_copy(vid_hbm.at[pl.ds(s * VC, VC)], vid_v)

    def flat_idx(q, carry):
        sl = pl.ds(q * L, L)
        v3 = vid_v[sl] * 3
        ix_v[sl] = v3
        iy_v[sl] = v3 + 1
        iz_v[sl] = v3 + 2
        return carry

    lax.fori_loop(0, VC // L, flat_idx, 0)

    for idx, dst in ((ix_v, px_v), (iy_v, py_v), (iz_v, pz_v)):
        for j in range(VB):
            sl = pl.ds(j * 128, 128)
            pltpu.async_copy(tet_hbm.at[idx.at[sl]], dst.at[sl], sem)
    for idx, dst in ((ix_v, px_v), (iy_v, py_v), (iz_v, pz_v)):
        for j in range(VB):
            sl = pl.ds(j * 128, 128)
            pltpu.make_async_copy(tet_hbm.at[idx.at[sl]], dst.at[sl], sem).wait()

    for sh, src in ((spx, px_v), (spy, py_v), (spz, pz_v)):
        pltpu.sync_copy(src, sh.at[pl.ds(s * VC, VC)])

    @pl.when(c == 0)
    def _():
        for dst, src in ((px_o, px_v), (py_o, py_v), (pz_o, pz_v)):
            pltpu.sync_copy(src, dst.at[pl.ds(s * VC, VC)])

    for fh, fv in ((f0_hbm, f0_v), (f1_hbm, f1_v), (f2_hbm, f2_v)):
        pltpu.sync_copy(fh.at[pl.ds(s * FC, FC)], fv)

    plsc.subcore_barrier()

    gathers = ((b0_v, (v0x, v0y, v0z)),
               (b1_v, (v1x, v1y, v1z)),
               (b2_v, (v2x, v2y, v2z)))

    def face_batch(b, carry):
        for fv, bv in ((f0_v, b0_v), (f1_v, b1_v), (f2_v, b2_v)):
            for q in range(128 // L):
                bv[pl.ds(q * L, L)] = fv[pl.ds(b * 128 + q * L, L)]
        for bv, dsts in gathers:
            for sh, dst in zip((spx, spy, spz), dsts):
                pltpu.async_copy(sh.at[bv], dst, sem)
        for bv, dsts in gathers:
            for sh, dst in zip((spx, spy, spz), dsts):
                pltpu.make_async_copy(sh.at[bv], dst, sem).wait()
        for g in range(128 // L):
            sl = pl.ds(g * L, L)
            ax, ay, az = v0x[sl], v0y[sl], v0z[sl]
            e1x, e1y, e1z = v1x[sl] - ax, v1y[sl] - ay, v1z[sl] - az
            e2x, e2y, e2z = v2x[sl] - ax, v2y[sl] - ay, v2z[sl] - az
            cx_v[sl] = e1y * e2z - e1z * e2y
            cy_v[sl] = e1z * e2x - e1x * e2z
            cz_v[sl] = e1x * e2y - e1y * e2x
        for bv in (b0_v, b1_v, b2_v):
            for sh, src in ((sax, cx_v), (say, cy_v), (saz, cz_v)):
                pltpu.sync_copy(src, sh.at[bv], add=True)
        return carry

    lax.fori_loop(0, FB, face_batch, 0)
    plsc.subcore_barrier()

    wid = c * NS + s
    base = wid * VC3
    sl3 = pl.ds(base, VC3)
    for sh, buf, dst in ((sax, ax_v, nx_o), (say, ay_v, ny_o), (saz, az_v, nz_o)):
        pltpu.sync_copy(sh.at[sl3], buf)
        pltpu.sync_copy(buf, dst.at[sl3])


def _norm_tc(ax, ay, az, ox, oy, oz):
    x, y, z = ax[...], ay[...], az[...]
    d = x * x + y * y + z * z
    ok = d > 1e-20
    n = jnp.maximum(jnp.sqrt(d), 1e-12)
    ox[...] = jnp.where(ok, x / n, 0.0)
    oy[...] = jnp.where(ok, y / n, 0.0)
    oz[...] = jnp.where(ok, z / n, 1.0)


@jax.jit
def kernel(tet_v, surface_vid, surface_f):
    tet_flat = tet_v.reshape(-1)
    vid = surface_vid.astype(jnp.int32)
    vid = jnp.concatenate([vid, jnp.zeros((NV_PAD - NV,), jnp.int32)])

    f32i = surface_f.astype(jnp.int32)
    pad = jnp.full((NF_PAD - NF,), NV, jnp.int32)
    f0 = jnp.concatenate([f32i[:, 0], pad])
    f1 = jnp.concatenate([f32i[:, 1], pad])
    f2 = jnp.concatenate([f32i[:, 2], pad])

    zer = jnp.zeros((VC,), jnp.float32)

    plane = jax.ShapeDtypeStruct((NV_PAD,), jnp.float32)
    vmemf = lambda n: pltpu.VMEM((n,), jnp.float32)
    vmemi = lambda n: pltpu.VMEM((n,), jnp.int32)
    run = pl.kernel(
        _body,
        out_type=(plane,) * 6,
        mesh=plsc.VectorSubcoreMesh(core_axis_name="c", subcore_axis_name="s"),
        scratch_types=[
            vmemi(VC), vmemi(VC), vmemi(VC), vmemi(VC),
            vmemf(VC), vmemf(VC), vmemf(VC),
            vmemi(FC), vmemi(FC), vmemi(FC),
            vmemi(128), vmemi(128), vmemi(128),
            vmemf(128), vmemf(128), vmemf(128),
            vmemf(128), vmemf(128), vmemf(128),
            vmemf(128), vmemf(128), vmemf(128),
            vmemf(128), vmemf(128), vmemf(128),
            vmemf(VC3), vmemf(VC3), vmemf(VC3),
            pltpu.VMEM_SHARED((NV_PAD,), jnp.float32),
            pltpu.VMEM_SHARED((NV_PAD,), jnp.float32),
            pltpu.VMEM_SHARED((NV_PAD,), jnp.float32),
            pltpu.VMEM_SHARED((NV_PAD,), jnp.float32),
            pltpu.VMEM_SHARED((NV_PAD,), jnp.float32),
            pltpu.VMEM_SHARED((NV_PAD,), jnp.float32),
            pltpu.SemaphoreType.DMA,
        ],
    )
    px, py, pz, ax, ay, az = run(tet_flat, vid, f0, f1, f2, zer)

    blk = jax.ShapeDtypeStruct((NV_PAD // 128, 128), jnp.float32)
    nx, ny, nz = pl.pallas_call(
        _norm_tc,
        out_shape=(blk,) * 3,
    )(ax.reshape(NV_PAD // 128, 128),
      ay.reshape(NV_PAD // 128, 128),
      az.reshape(NV_PAD // 128, 128))

    v_pos = jnp.stack([px[:NV], py[:NV], pz[:NV]], axis=1)
    v_nrm = jnp.stack([nx.reshape(-1)[:NV],
                       ny.reshape(-1)[:NV],
                       nz.reshape(-1)[:NV]], axis=1)
    return v_pos, v_nrm


# speedup vs baseline: 9.0404x; 1.1570x over previous
import jax
import jax.numpy as jnp
from jax import lax
from jax.experimental import pallas as pl
from jax.experimental.pallas import tpu as pltpu
from jax.experimental.pallas import tpu_sc as plsc

N_TET_V = 100000
NV = 50000
NF = 100000

NS = 16
L = 16

VC = 3200
NV_PAD = NS * VC
VB = VC // 128

FC = 3200
NF_PAD = 32 * FC
FB = FC // 128


def _body(tet_hbm, vid_hbm, f0_hbm, f1_hbm, f2_hbm, zer_hbm,
          px_o, py_o, pz_o, a0x_o, a0y_o, a0z_o, a1x_o, a1y_o, a1z_o,
          vid_v, ix_v, iy_v, iz_v, px_v, py_v, pz_v,
          f0_v, f1_v, f2_v, b0_v, b1_v, b2_v,
          v0x, v0y, v0z, v1x, v1y, v1z, v2x, v2y, v2z,
          cx_v, cy_v, cz_v, ax_v, ay_v, az_v,
          spx, spy, spz, sax, say, saz, sem):
    c = lax.axis_index("c")
    s = lax.axis_index("s")

    for sh in (sax, say, saz):
        pltpu.sync_copy(zer_hbm, sh.at[pl.ds(s * VC, VC)])

    pltpu.sync_copy(vid_hbm.at[pl.ds(s * VC, VC)], vid_v)

    def flat_idx(q, carry):
        sl = pl.ds(q * L, L)
        v3 = vid_v[sl] * 3
        ix_v[sl] = v3
        iy_v[sl] = v3 + 1
        iz_v[sl] = v3 + 2
        return carry

    lax.fori_loop(0, VC // L, flat_idx, 0)

    for idx, dst in ((ix_v, px_v), (iy_v, py_v), (iz_v, pz_v)):
        for j in range(VB):
            sl = pl.ds(j * 128, 128)
            pltpu.async_copy(tet_hbm.at[idx.at[sl]], dst.at[sl], sem)
    for idx, dst in ((ix_v, px_v), (iy_v, py_v), (iz_v, pz_v)):
        for j in range(VB):
            sl = pl.ds(j * 128, 128)
            pltpu.make_async_copy(tet_hbm.at[idx.at[sl]], dst.at[sl], sem).wait()

    for sh, src in ((spx, px_v), (spy, py_v), (spz, pz_v)):
        pltpu.sync_copy(src, sh.at[pl.ds(s * VC, VC)])

    @pl.when(c == 0)
    def _():
        for dst, src in ((px_o, px_v), (py_o, py_v), (pz_o, pz_v)):
            pltpu.sync_copy(src, dst.at[pl.ds(s * VC, VC)])

    wid = c * NS + s
    for fh, fv in ((f0_hbm, f0_v), (f1_hbm, f1_v), (f2_hbm, f2_v)):
        pltpu.sync_copy(fh.at[pl.ds(wid * FC, FC)], fv)

    plsc.subcore_barrier()

    gathers = ((b0_v, (v0x, v0y, v0z)),
               (b1_v, (v1x, v1y, v1z)),
               (b2_v, (v2x, v2y, v2z)))

    def face_batch(b, carry):
        for fv, bv in ((f0_v, b0_v), (f1_v, b1_v), (f2_v, b2_v)):
            for q in range(128 // L):
                bv[pl.ds(q * L, L)] = fv[pl.ds(b * 128 + q * L, L)]
        for bv, dsts in gathers:
            for sh, dst in zip((spx, spy, spz), dsts):
                pltpu.async_copy(sh.at[bv], dst, sem)
        for bv, dsts in gathers:
            for sh, dst in zip((spx, spy, spz), dsts):
                pltpu.make_async_copy(sh.at[bv], dst, sem).wait()
        for g in range(128 // L):
            sl = pl.ds(g * L, L)
            ax, ay, az = v0x[sl], v0y[sl], v0z[sl]
            e1x, e1y, e1z = v1x[sl] - ax, v1y[sl] - ay, v1z[sl] - az
            e2x, e2y, e2z = v2x[sl] - ax, v2y[sl] - ay, v2z[sl] - az
            cx_v[sl] = e1y * e2z - e1z * e2y
            cy_v[sl] = e1z * e2x - e1x * e2z
            cz_v[sl] = e1x * e2y - e1y * e2x
        for bv in (b0_v, b1_v, b2_v):
            for sh, src in ((sax, cx_v), (say, cy_v), (saz, cz_v)):
                pltpu.sync_copy(src, sh.at[bv], add=True)
        return carry

    lax.fori_loop(0, FB, face_batch, 0)
    plsc.subcore_barrier()

    sl3 = pl.ds(s * VC, VC)

    @pl.when(c == 0)
    def _():
        for sh, buf, dst in ((sax, ax_v, a0x_o), (say, ay_v, a0y_o), (saz, az_v, a0z_o)):
            pltpu.sync_copy(sh.at[sl3], buf)
            pltpu.sync_copy(buf, dst.at[sl3])

    @pl.when(c == 1)
    def _():
        for sh, buf, dst in ((sax, ax_v, a1x_o), (say, ay_v, a1y_o), (saz, az_v, a1z_o)):
            pltpu.sync_copy(sh.at[sl3], buf)
            pltpu.sync_copy(buf, dst.at[sl3])


def _norm_tc(a0x, a0y, a0z, a1x, a1y, a1z, ox, oy, oz):
    x = a0x[...] + a1x[...]
    y = a0y[...] + a1y[...]
    z = a0z[...] + a1z[...]
    d = x * x + y * y + z * z
    ok = d > 1e-20
    n = jnp.maximum(jnp.sqrt(d), 1e-12)
    ox[...] = jnp.where(ok, x / n, 0.0)
    oy[...] = jnp.where(ok, y / n, 0.0)
    oz[...] = jnp.where(ok, z / n, 1.0)


@jax.jit
def kernel(tet_v, surface_vid, surface_f):
    tet_flat = tet_v.reshape(-1)
    vid = surface_vid.astype(jnp.int32)
    vid = jnp.concatenate([vid, jnp.zeros((NV_PAD - NV,), jnp.int32)])

    f32i = surface_f.astype(jnp.int32)
    pad = jnp.full((NF_PAD - NF,), NV, jnp.int32)
    f0 = jnp.concatenate([f32i[:, 0], pad])
    f1 = jnp.concatenate([f32i[:, 1], pad])
    f2 = jnp.concatenate([f32i[:, 2], pad])

    zer = jnp.zeros((VC,), jnp.float32)

    plane = jax.ShapeDtypeStruct((NV_PAD,), jnp.float32)
    vmemf = lambda n: pltpu.VMEM((n,), jnp.float32)
    vmemi = lambda n: pltpu.VMEM((n,), jnp.int32)
    run = pl.kernel(
        _body,
        out_type=(plane,) * 9,
        mesh=plsc.VectorSubcoreMesh(core_axis_name="c", subcore_axis_name="s"),
        scratch_types=[
            vmemi(VC), vmemi(VC), vmemi(VC), vmemi(VC),
            vmemf(VC), vmemf(VC), vmemf(VC),
            vmemi(FC), vmemi(FC), vmemi(FC),
            vmemi(128), vmemi(128), vmemi(128),
            vmemf(128), vmemf(128), vmemf(128),
            vmemf(128), vmemf(128), vmemf(128),
            vmemf(128), vmemf(128), vmemf(128),
            vmemf(128), vmemf(128), vmemf(128),
            vmemf(VC), vmemf(VC), vmemf(VC),
            pltpu.VMEM_SHARED((NV_PAD,), jnp.float32),
            pltpu.VMEM_SHARED((NV_PAD,), jnp.float32),
            pltpu.VMEM_SHARED((NV_PAD,), jnp.float32),
            pltpu.VMEM_SHARED((NV_PAD,), jnp.float32),
            pltpu.VMEM_SHARED((NV_PAD,), jnp.float32),
            pltpu.VMEM_SHARED((NV_PAD,), jnp.float32),
            pltpu.SemaphoreType.DMA,
        ],
    )
    px, py, pz, a0x, a0y, a0z, a1x, a1y, a1z = run(tet_flat, vid, f0, f1, f2, zer)

    blk = jax.ShapeDtypeStruct((NV_PAD // 128, 128), jnp.float32)
    nx, ny, nz = pl.pallas_call(
        _norm_tc,
        out_shape=(blk,) * 3,
    )(*(a.reshape(NV_PAD // 128, 128) for a in (a0x, a0y, a0z, a1x, a1y, a1z)))

    v_pos = jnp.stack([px[:NV], py[:NV], pz[:NV]], axis=1)
    v_nrm = jnp.stack([nx.reshape(-1)[:NV],
                       ny.reshape(-1)[:NV],
                       nz.reshape(-1)[:NV]], axis=1)
    return v_pos, v_nrm


# speedup vs baseline: 18.1835x; 2.0114x over previous
import jax
import jax.numpy as jnp
from jax import lax
from jax.experimental import pallas as pl
from jax.experimental.pallas import tpu as pltpu
from jax.experimental.pallas import tpu_sc as plsc

N_TET_V = 100000
NV = 50000
NF = 100000

NS = 16
L = 16

TC_ = 6256
NT_PAD = NS * TC_

VC = 3200
NV_PAD = NS * VC
VB = VC // 128

FC = 3200
NF_PAD = 32 * FC
FB = FC // 128


def _body(tx_hbm, ty_hbm, tz_hbm, vid_hbm, f0_hbm, f1_hbm, f2_hbm, zer_hbm,
          px_o, py_o, pz_o, a0x_o, a0y_o, a0z_o, a1x_o, a1y_o, a1z_o,
          tb_v, vid_v, px_v, py_v, pz_v,
          f0_v, f1_v, f2_v, b0_v, b1_v, b2_v,
          v0x, v0y, v0z, v1x, v1y, v1z, v2x, v2y, v2z,
          cx_v, cy_v, cz_v, ax_v, ay_v, az_v,
          stx, sty, stz, spx, spy, spz, sax, say, saz,
          sem, sem2):
    c = lax.axis_index("c")
    s = lax.axis_index("s")

    for sh in (sax, say, saz):
        pltpu.sync_copy(zer_hbm, sh.at[pl.ds(s * VC, VC)])

    slt = pl.ds(s * TC_, TC_)
    for th, st in ((tx_hbm, stx), (ty_hbm, sty), (tz_hbm, stz)):
        pltpu.sync_copy(th.at[slt], tb_v)
        pltpu.sync_copy(tb_v, st.at[slt])

    pltpu.sync_copy(vid_hbm.at[pl.ds(s * VC, VC)], vid_v)

    wid = c * NS + s
    for fh, fv in ((f0_hbm, f0_v), (f1_hbm, f1_v), (f2_hbm, f2_v)):
        pltpu.sync_copy(fh.at[pl.ds(wid * FC, FC)], fv)

    plsc.subcore_barrier()

    for st, dst in ((stx, px_v), (sty, py_v), (stz, pz_v)):
        for j in range(VB):
            sl = pl.ds(j * 128, 128)
            pltpu.async_copy(st.at[vid_v.at[sl]], dst.at[sl], sem)
    for st, dst in ((stx, px_v), (sty, py_v), (stz, pz_v)):
        for j in range(VB):
            sl = pl.ds(j * 128, 128)
            pltpu.make_async_copy(st.at[vid_v.at[sl]], dst.at[sl], sem).wait()

    for sh, src in ((spx, px_v), (spy, py_v), (spz, pz_v)):
        pltpu.sync_copy(src, sh.at[pl.ds(s * VC, VC)])

    @pl.when(c == 0)
    def _():
        for dst, src in ((px_o, px_v), (py_o, py_v), (pz_o, pz_v)):
            pltpu.sync_copy(src, dst.at[pl.ds(s * VC, VC)])

    plsc.subcore_barrier()

    gathers = ((b0_v, (v0x, v0y, v0z)),
               (b1_v, (v1x, v1y, v1z)),
               (b2_v, (v2x, v2y, v2z)))

    def face_batch(b, carry):
        for fv, bv in ((f0_v, b0_v), (f1_v, b1_v), (f2_v, b2_v)):
            for q in range(128 // L):
                bv[pl.ds(q * L, L)] = fv[pl.ds(b * 128 + q * L, L)]
        for bv, dsts in gathers:
            for sh, dst in zip((spx, spy, spz), dsts):
                pltpu.async_copy(sh.at[bv], dst, sem)
        for bv, dsts in gathers:
            for sh, dst in zip((spx, spy, spz), dsts):
                pltpu.make_async_copy(sh.at[bv], dst, sem).wait()
        for g in range(128 // L):
            sl = pl.ds(g * L, L)
            so = pl.ds(b * 128 + g * L, L)
            ax, ay, az = v0x[sl], v0y[sl], v0z[sl]
            e1x, e1y, e1z = v1x[sl] - ax, v1y[sl] - ay, v1z[sl] - az
            e2x, e2y, e2z = v2x[sl] - ax, v2y[sl] - ay, v2z[sl] - az
            cx_v[so] = e1y * e2z - e1z * e2y
            cy_v[so] = e1z * e2x - e1x * e2z
            cz_v[so] = e1x * e2y - e1y * e2x
        slb = pl.ds(b * 128, 128)
        for bv in (b0_v, b1_v, b2_v):
            for sh, src in ((sax, cx_v), (say, cy_v), (saz, cz_v)):
                pltpu.async_copy(src.at[slb], sh.at[bv], sem2, add=True)
        return carry

    lax.fori_loop(0, FB, face_batch, 0)

    def drain(i, carry):
        pltpu.make_async_copy(cx_v.at[pl.ds(0, 128)], sax.at[b0_v], sem2).wait()
        return carry

    lax.fori_loop(0, FB * 9, drain, 0)
    plsc.subcore_barrier()

    sl3 = pl.ds(s * VC, VC)

    @pl.when(c == 0)
    def _():
        for sh, buf, dst in ((sax, ax_v, a0x_o), (say, ay_v, a0y_o), (saz, az_v, a0z_o)):
            pltpu.sync_copy(sh.at[sl3], buf)
            pltpu.sync_copy(buf, dst.at[sl3])

    @pl.when(c == 1)
    def _():
        for sh, buf, dst in ((sax, ax_v, a1x_o), (say, ay_v, a1y_o), (saz, az_v, a1z_o)):
            pltpu.sync_copy(sh.at[sl3], buf)
            pltpu.sync_copy(buf, dst.at[sl3])


def _norm_tc(a0x, a0y, a0z, a1x, a1y, a1z, ox, oy, oz):
    x = a0x[...] + a1x[...]
    y = a0y[...] + a1y[...]
    z = a0z[...] + a1z[...]
    d = x * x + y * y + z * z
    ok = d > 1e-20
    n = jnp.maximum(jnp.sqrt(d), 1e-12)
    ox[...] = jnp.where(ok, x / n, 0.0)
    oy[...] = jnp.where(ok, y / n, 0.0)
    oz[...] = jnp.where(ok, z / n, 1.0)


@jax.jit
def kernel(tet_v, surface_vid, surface_f):
    tpad = jnp.zeros((NT_PAD - N_TET_V,), jnp.float32)
    tx = jnp.concatenate([tet_v[:, 0], tpad])
    ty = jnp.concatenate([tet_v[:, 1], tpad])
    tz = jnp.concatenate([tet_v[:, 2], tpad])

    vid = surface_vid.astype(jnp.int32)
    vid = jnp.concatenate([vid, jnp.zeros((NV_PAD - NV,), jnp.int32)])

    f32i = surface_f.astype(jnp.int32)
    pad = jnp.full((NF_PAD - NF,), NV, jnp.int32)
    f0 = jnp.concatenate([f32i[:, 0], pad])
    f1 = jnp.concatenate([f32i[:, 1], pad])
    f2 = jnp.concatenate([f32i[:, 2], pad])

    zer = jnp.zeros((VC,), jnp.float32)

    plane = jax.ShapeDtypeStruct((NV_PAD,), jnp.float32)
    vmemf = lambda n: pltpu.VMEM((n,), jnp.float32)
    vmemi = lambda n: pltpu.VMEM((n,), jnp.int32)
    shmf = lambda n: pltpu.VMEM_SHARED((n,), jnp.float32)
    run = pl.kernel(
        _body,
        out_type=(plane,) * 9,
        mesh=plsc.VectorSubcoreMesh(core_axis_name="c", subcore_axis_name="s"),
        scratch_types=[
            vmemf(TC_), vmemi(VC),
            vmemf(VC), vmemf(VC), vmemf(VC),
            vmemi(FC), vmemi(FC), vmemi(FC),
            vmemi(128), vmemi(128), vmemi(128),
            vmemf(128), vmemf(128), vmemf(128),
            vmemf(128), vmemf(128), vmemf(128),
            vmemf(128), vmemf(128), vmemf(128),
            vmemf(FC), vmemf(FC), vmemf(FC),
            vmemf(VC), vmemf(VC), vmemf(VC),
            shmf(NT_PAD), shmf(NT_PAD), shmf(NT_PAD),
            shmf(NV_PAD), shmf(NV_PAD), shmf(NV_PAD),
            shmf(NV_PAD), shmf(NV_PAD), shmf(NV_PAD),
            pltpu.SemaphoreType.DMA,
            pltpu.SemaphoreType.DMA,
        ],
    )
    px, py, pz, a0x, a0y, a0z, a1x, a1y, a1z = run(
        tx, ty, tz, vid, f0, f1, f2, zer)

    blk = jax.ShapeDtypeStruct((NV_PAD // 128, 128), jnp.float32)
    nx, ny, nz = pl.pallas_call(
        _norm_tc,
        out_shape=(blk,) * 3,
    )(*(a.reshape(NV_PAD // 128, 128) for a in (a0x, a0y, a0z, a1x, a1y, a1z)))

    v_pos = jnp.stack([px[:NV], py[:NV], pz[:NV]], axis=1)
    v_nrm = jnp.stack([nx.reshape(-1)[:NV],
                       ny.reshape(-1)[:NV],
                       nz.reshape(-1)[:NV]], axis=1)
    return v_pos, v_nrm
